# baseline (speedup 1.0000x reference)
import functools

import jax
import jax.numpy as jnp
from jax import lax
from jax.experimental import pallas as pl
from jax.experimental.pallas import tpu as pltpu
from jax.experimental.pallas import tpu_sc as plsc

LENGTH = 2048
N_PAIRS = 16
FLAT = 2 * N_PAIRS

_mesh = plsc.VectorSubcoreMesh(
    core_axis_name="c", subcore_axis_name="s", num_cores=1, num_subcores=1
)


@functools.partial(
    pl.kernel,
    mesh=_mesh,
    out_type=jax.ShapeDtypeStruct((FLAT,), jnp.int32),
    scratch_types=[pltpu.VMEM((FLAT,), jnp.int32)],
)
def _trunc_sc(idx_hbm, out_hbm, scratch):
    cid = lax.axis_index("c")
    sid = lax.axis_index("s")

    @pl.when(jnp.logical_and(cid == 0, sid == 0))
    def _():
        pltpu.sync_copy(idx_hbm, scratch)
        lane = lax.iota(jnp.int32, 16)
        even = lane - (lane & 1)
        for i in range(FLAT // 16):
            x = scratch[pl.ds(16 * i, 16)]
            b = x.at[even].get(mode="promise_in_bounds")
            scratch[pl.ds(16 * i, 16)] = jnp.minimum(x, b + LENGTH)
        pltpu.sync_copy(scratch, out_hbm)


_N_COPY_CHUNKS = 4


def _copy_body(x_ref, o_ref, sem):
    rows = x_ref.shape[0]
    chunk = rows // _N_COPY_CHUNKS
    for k in range(_N_COPY_CHUNKS):
        pltpu.make_async_copy(
            x_ref.at[pl.ds(k * chunk, chunk)],
            o_ref.at[pl.ds(k * chunk, chunk)],
            sem,
        ).start()
    for k in range(_N_COPY_CHUNKS):
        pltpu.make_async_copy(
            x_ref.at[pl.ds(k * chunk, chunk)],
            o_ref.at[pl.ds(k * chunk, chunk)],
            sem,
        ).wait()


def _tc_copy(values):
    return pl.pallas_call(
        _copy_body,
        in_specs=[pl.BlockSpec(memory_space=pl.ANY)],
        out_specs=pl.BlockSpec(memory_space=pl.ANY),
        scratch_shapes=[pltpu.SemaphoreType.DMA],
        out_shape=jax.ShapeDtypeStruct(values.shape, values.dtype),
    )(values)


def kernel(values, indices):
    vals_out = _tc_copy(values)
    out = _trunc_sc(indices.reshape(FLAT))
    return (vals_out, out.reshape(N_PAIRS, 2))


# speedup vs baseline: 41.0494x; 41.0494x over previous
import functools

import jax
import jax.numpy as jnp
from jax import lax
from jax.experimental import pallas as pl
from jax.experimental.pallas import tpu as pltpu
from jax.experimental.pallas import tpu_sc as plsc

LENGTH = 2048
N_PAIRS = 16
FLAT = 2 * N_PAIRS

_mesh = plsc.VectorSubcoreMesh(
    core_axis_name="c", subcore_axis_name="s", num_cores=1, num_subcores=1
)


@functools.partial(
    pl.kernel,
    mesh=_mesh,
    out_type=jax.ShapeDtypeStruct((FLAT,), jnp.int32),
    scratch_types=[pltpu.VMEM((FLAT,), jnp.int32)],
)
def _trunc_sc(idx_hbm, out_hbm, scratch):
    cid = lax.axis_index("c")
    sid = lax.axis_index("s")

    @pl.when(jnp.logical_and(cid == 0, sid == 0))
    def _():
        pltpu.sync_copy(idx_hbm, scratch)
        lane = lax.iota(jnp.int32, 16)
        even = lane - (lane & 1)
        for i in range(FLAT // 16):
            x = scratch[pl.ds(16 * i, 16)]
            b = x.at[even].get(mode="promise_in_bounds")
            scratch[pl.ds(16 * i, 16)] = jnp.minimum(x, b + LENGTH)
        pltpu.sync_copy(scratch, out_hbm)


_COPY_BLOCK = 2048


def _copy_body(x_ref, o_ref):
    o_ref[...] = x_ref[...]


def _tc_copy(values):
    rows, cols = values.shape
    return pl.pallas_call(
        _copy_body,
        grid=(rows // _COPY_BLOCK,),
        in_specs=[pl.BlockSpec((_COPY_BLOCK, cols), lambda i: (i, 0))],
        out_specs=pl.BlockSpec((_COPY_BLOCK, cols), lambda i: (i, 0)),
        out_shape=jax.ShapeDtypeStruct(values.shape, values.dtype),
    )(values)


def kernel(values, indices):
    vals_out = _tc_copy(values)
    out = _trunc_sc(indices.reshape(FLAT))
    return (vals_out, out.reshape(N_PAIRS, 2))


# speedup vs baseline: 41.1168x; 1.0016x over previous
import functools

import jax
import jax.numpy as jnp
from jax import lax
from jax.experimental import pallas as pl
from jax.experimental.pallas import tpu as pltpu
from jax.experimental.pallas import tpu_sc as plsc

LENGTH = 2048
N_PAIRS = 16
FLAT = 2 * N_PAIRS

_mesh = plsc.ScalarSubcoreMesh(axis_name="c", num_cores=1)


@functools.partial(
    pl.kernel,
    mesh=_mesh,
    out_type=jax.ShapeDtypeStruct((FLAT,), jnp.int32),
    scratch_types=[pltpu.SMEM((FLAT,), jnp.int32)],
)
def _trunc_sc(idx_hbm, out_hbm, scratch):
    cid = lax.axis_index("c")

    @pl.when(cid == 0)
    def _():
        pltpu.sync_copy(idx_hbm, scratch)
        for i in range(N_PAIRS):
            b = scratch[2 * i]
            e = scratch[2 * i + 1]
            scratch[2 * i + 1] = jnp.minimum(b + LENGTH, e)
        pltpu.sync_copy(scratch, out_hbm)


_COPY_BLOCK = 2048


def _copy_body(x_ref, o_ref):
    o_ref[...] = x_ref[...]


def _tc_copy(values):
    rows, cols = values.shape
    return pl.pallas_call(
        _copy_body,
        grid=(rows // _COPY_BLOCK,),
        in_specs=[pl.BlockSpec((_COPY_BLOCK, cols), lambda i: (i, 0))],
        out_specs=pl.BlockSpec((_COPY_BLOCK, cols), lambda i: (i, 0)),
        out_shape=jax.ShapeDtypeStruct(values.shape, values.dtype),
    )(values)


def kernel(values, indices):
    vals_out = _tc_copy(values)
    out = _trunc_sc(indices.reshape(FLAT))
    return (vals_out, out.reshape(N_PAIRS, 2))
